# baseline (speedup 1.0000x reference)
import numpy as np
import jax
import jax.numpy as jnp
from jax.experimental import pallas as pl

_ROWS = 128
_VOCAB = 100000


def _threefry2x32(k0, k1, x0, x1):
    def rotl(x, r):
        return ((x << np.uint32(r)) | (x >> np.uint32(32 - r))).astype(np.uint32)

    ks0 = np.uint32(k0)
    ks1 = np.uint32(k1)
    ks2 = np.uint32(ks0 ^ ks1 ^ np.uint32(0x1BD11BDA))
    x0 = (x0 + ks0).astype(np.uint32)
    x1 = (x1 + ks1).astype(np.uint32)
    rots = [(13, 15, 26, 6), (17, 29, 16, 24)]
    inject = [(ks1, ks2), (ks2, ks0), (ks0, ks1), (ks1, ks2), (ks2, ks0)]
    for i in range(5):
        for r in rots[i % 2]:
            x0 = (x0 + x1).astype(np.uint32)
            x1 = rotl(x1, r)
            x1 = (x1 ^ x0).astype(np.uint32)
        a, b = inject[i]
        x0 = (x0 + a).astype(np.uint32)
        x1 = (x1 + b + np.uint32(i + 1)).astype(np.uint32)
    return x0, x1


def _gumbel_table():
    n = _ROWS * _VOCAB
    o0, o1 = _threefry2x32(
        0, 42, np.zeros(n, dtype=np.uint32), np.arange(n, dtype=np.uint32)
    )
    bits = (o0 ^ o1).astype(np.uint32)
    fb = ((bits >> np.uint32(9)) | np.uint32(0x3F800000)).astype(np.uint32)
    u = fb.view(np.float32).astype(np.float64) - 1.0
    e = (-np.log1p(-u)).astype(np.float32)
    e = np.maximum(e, np.float32(1e-10))
    g = (-np.log(e.astype(np.float64))).astype(np.float32)
    return g.reshape(_ROWS, _VOCAB)


_GUMBEL = _gumbel_table()


_RB = 8
_LANES = 128
_K = 4
_CHUNK = _K * _LANES
_NITER = 195
_TAIL0 = _NITER * _CHUNK
_BIG = 2**30


def _sample_body(t_ref, x_ref, g_ref, o_ref):
    t = t_ref[...]
    lane = jax.lax.broadcasted_iota(jnp.int32, (_RB, _LANES), 1)
    neg_inf = jnp.full((_RB, _LANES), -jnp.inf, jnp.float32)
    zeros_i = jnp.zeros((_RB, _LANES), jnp.int32)

    def body(i, st):
        ms, ids = list(st[0]), list(st[1])
        base = i * _CHUNK
        for k in range(_K):
            off = pl.multiple_of(base + k * _LANES, _LANES)
            s = x_ref[:, pl.ds(off, _LANES)] / t + g_ref[:, pl.ds(off, _LANES)]
            gt = s > ms[k]
            ms[k] = jnp.where(gt, s, ms[k])
            ids[k] = jnp.where(gt, i, ids[k])
        return (tuple(ms), tuple(ids))

    st = jax.lax.fori_loop(
        0, _NITER, body, ((neg_inf,) * _K, (zeros_i,) * _K)
    )
    best_v, best_c = None, None
    for k in range(_K):
        v = st[0][k]
        c = st[1][k] * _CHUNK + (k * _LANES) + lane
        if best_v is None:
            best_v, best_c = v, c
        else:
            take = (v > best_v) | ((v == best_v) & (c < best_c))
            best_v = jnp.where(take, v, best_v)
            best_c = jnp.where(take, c, best_c)
    row_v = jnp.max(best_v, axis=1, keepdims=True)
    row_c = jnp.min(jnp.where(best_v == row_v, best_c, _BIG), axis=1,
                    keepdims=True)
    s_t = x_ref[:, _TAIL0:_VOCAB] / t + g_ref[:, _TAIL0:_VOCAB]
    col_t = _TAIL0 + jax.lax.broadcasted_iota(jnp.int32, s_t.shape, 1)
    row_vt = jnp.max(s_t, axis=1, keepdims=True)
    row_ct = jnp.min(jnp.where(s_t == row_vt, col_t, _BIG), axis=1,
                     keepdims=True)
    take_t = (row_vt > row_v) | ((row_vt == row_v) & (row_ct < row_c))
    o_ref[...] = jnp.where(take_t, row_ct, row_c)


def kernel(logits, temperatures):
    rb = 8
    grid = (_ROWS // rb,)
    out = pl.pallas_call(
        _sample_body,
        grid=grid,
        in_specs=[
            pl.BlockSpec((rb, 1), lambda i: (i, 0)),
            pl.BlockSpec((rb, _VOCAB), lambda i: (i, 0)),
            pl.BlockSpec((rb, _VOCAB), lambda i: (i, 0)),
        ],
        out_specs=pl.BlockSpec((rb, 1), lambda i: (i, 0)),
        out_shape=jax.ShapeDtypeStruct((_ROWS, 1), jnp.int32),
    )(temperatures[:, None], logits, jnp.asarray(_GUMBEL))
    return out.reshape(_ROWS)


# speedup vs baseline: 3.3899x; 3.3899x over previous
import numpy as np
import jax
import jax.numpy as jnp
from jax.experimental import pallas as pl

_ROWS = 128
_VOCAB = 100000


def _threefry2x32(k0, k1, x0, x1):
    def rotl(x, r):
        return ((x << np.uint32(r)) | (x >> np.uint32(32 - r))).astype(np.uint32)

    ks0 = np.uint32(k0)
    ks1 = np.uint32(k1)
    ks2 = np.uint32(ks0 ^ ks1 ^ np.uint32(0x1BD11BDA))
    x0 = (x0 + ks0).astype(np.uint32)
    x1 = (x1 + ks1).astype(np.uint32)
    rots = [(13, 15, 26, 6), (17, 29, 16, 24)]
    inject = [(ks1, ks2), (ks2, ks0), (ks0, ks1), (ks1, ks2), (ks2, ks0)]
    for i in range(5):
        for r in rots[i % 2]:
            x0 = (x0 + x1).astype(np.uint32)
            x1 = rotl(x1, r)
            x1 = (x1 ^ x0).astype(np.uint32)
        a, b = inject[i]
        x0 = (x0 + a).astype(np.uint32)
        x1 = (x1 + b + np.uint32(i + 1)).astype(np.uint32)
    return x0, x1


def _gumbel_table():
    n = _ROWS * _VOCAB
    o0, o1 = _threefry2x32(
        0, 42, np.zeros(n, dtype=np.uint32), np.arange(n, dtype=np.uint32)
    )
    bits = (o0 ^ o1).astype(np.uint32)
    fb = ((bits >> np.uint32(9)) | np.uint32(0x3F800000)).astype(np.uint32)
    u = fb.view(np.float32).astype(np.float64) - 1.0
    e = (-np.log1p(-u)).astype(np.float32)
    e = np.maximum(e, np.float32(1e-10))
    g = (-np.log(e.astype(np.float64))).astype(np.float32)
    return g.reshape(_ROWS, _VOCAB)


_GUMBEL = _gumbel_table()


_RB = 8
_LANES = 128
_K = 4
_CHUNK = _K * _LANES
_NITER = 195
_TAIL0 = _NITER * _CHUNK
_BIG = 2**30


def _sample_body(t_ref, x_ref, g_ref, o_ref):
    l = x_ref[...] / t_ref[...]
    s = l + g_ref[...]
    col = jax.lax.broadcasted_iota(jnp.int32, s.shape, 1)
    s = jnp.where(col < _VOCAB, s, -jnp.inf)
    m = jnp.max(s, axis=1, keepdims=True)
    idx = jnp.min(jnp.where(s == m, col, jnp.int32(_BIG)), axis=1)
    o_ref[...] = idx[:, None]


def kernel(logits, temperatures):
    rb = 8
    grid = (_ROWS // rb,)
    out = pl.pallas_call(
        _sample_body,
        grid=grid,
        in_specs=[
            pl.BlockSpec((rb, 1), lambda i: (i, 0)),
            pl.BlockSpec((rb, _VOCAB), lambda i: (i, 0)),
            pl.BlockSpec((rb, _VOCAB), lambda i: (i, 0)),
        ],
        out_specs=pl.BlockSpec((rb, 1), lambda i: (i, 0)),
        out_shape=jax.ShapeDtypeStruct((_ROWS, 1), jnp.int32),
    )(temperatures[:, None], logits, jnp.asarray(_GUMBEL))
    return out.reshape(_ROWS)


# speedup vs baseline: 3.6212x; 1.0682x over previous
import numpy as np
import jax
import jax.numpy as jnp
from jax.experimental import pallas as pl

_ROWS = 128
_VOCAB = 100000


def _threefry2x32(k0, k1, x0, x1):
    def rotl(x, r):
        return ((x << np.uint32(r)) | (x >> np.uint32(32 - r))).astype(np.uint32)

    ks0 = np.uint32(k0)
    ks1 = np.uint32(k1)
    ks2 = np.uint32(ks0 ^ ks1 ^ np.uint32(0x1BD11BDA))
    x0 = (x0 + ks0).astype(np.uint32)
    x1 = (x1 + ks1).astype(np.uint32)
    rots = [(13, 15, 26, 6), (17, 29, 16, 24)]
    inject = [(ks1, ks2), (ks2, ks0), (ks0, ks1), (ks1, ks2), (ks2, ks0)]
    for i in range(5):
        for r in rots[i % 2]:
            x0 = (x0 + x1).astype(np.uint32)
            x1 = rotl(x1, r)
            x1 = (x1 ^ x0).astype(np.uint32)
        a, b = inject[i]
        x0 = (x0 + a).astype(np.uint32)
        x1 = (x1 + b + np.uint32(i + 1)).astype(np.uint32)
    return x0, x1


def _gumbel_table():
    n = _ROWS * _VOCAB
    o0, o1 = _threefry2x32(
        0, 42, np.zeros(n, dtype=np.uint32), np.arange(n, dtype=np.uint32)
    )
    bits = (o0 ^ o1).astype(np.uint32)
    fb = ((bits >> np.uint32(9)) | np.uint32(0x3F800000)).astype(np.uint32)
    u = fb.view(np.float32).astype(np.float64) - 1.0
    e = (-np.log1p(-u)).astype(np.float32)
    e = np.maximum(e, np.float32(1e-10))
    g = (-np.log(e.astype(np.float64))).astype(np.float32)
    return g.reshape(_ROWS, _VOCAB)


_GUMBEL = _gumbel_table()


_RB = 8
_LANES = 128
_K = 4
_CHUNK = _K * _LANES
_NITER = 195
_TAIL0 = _NITER * _CHUNK
_BIG = 2**30


def _sample_body(t_ref, x_ref, g_ref, o_ref):
    l = x_ref[...] / t_ref[...]
    s = l + g_ref[...]
    col = jax.lax.broadcasted_iota(jnp.int32, s.shape, 1)
    s = jnp.where(col < _VOCAB, s, -jnp.inf)
    idx = jnp.argmax(s, axis=1)
    o_ref[...] = idx[:, None].astype(jnp.int32)


def kernel(logits, temperatures):
    rb = 8
    grid = (_ROWS // rb,)
    out = pl.pallas_call(
        _sample_body,
        grid=grid,
        in_specs=[
            pl.BlockSpec((rb, 1), lambda i: (i, 0)),
            pl.BlockSpec((rb, _VOCAB), lambda i: (i, 0)),
            pl.BlockSpec((rb, _VOCAB), lambda i: (i, 0)),
        ],
        out_specs=pl.BlockSpec((rb, 1), lambda i: (i, 0)),
        out_shape=jax.ShapeDtypeStruct((_ROWS, 1), jnp.int32),
    )(temperatures[:, None], logits, jnp.asarray(_GUMBEL))
    return out.reshape(_ROWS)


# speedup vs baseline: 3.6909x; 1.0193x over previous
import numpy as np
import jax
import jax.numpy as jnp
from jax.experimental import pallas as pl

_ROWS = 128
_VOCAB = 100000


def _threefry2x32(k0, k1, x0, x1):
    def rotl(x, r):
        return ((x << np.uint32(r)) | (x >> np.uint32(32 - r))).astype(np.uint32)

    ks0 = np.uint32(k0)
    ks1 = np.uint32(k1)
    ks2 = np.uint32(ks0 ^ ks1 ^ np.uint32(0x1BD11BDA))
    x0 = (x0 + ks0).astype(np.uint32)
    x1 = (x1 + ks1).astype(np.uint32)
    rots = [(13, 15, 26, 6), (17, 29, 16, 24)]
    inject = [(ks1, ks2), (ks2, ks0), (ks0, ks1), (ks1, ks2), (ks2, ks0)]
    for i in range(5):
        for r in rots[i % 2]:
            x0 = (x0 + x1).astype(np.uint32)
            x1 = rotl(x1, r)
            x1 = (x1 ^ x0).astype(np.uint32)
        a, b = inject[i]
        x0 = (x0 + a).astype(np.uint32)
        x1 = (x1 + b + np.uint32(i + 1)).astype(np.uint32)
    return x0, x1


def _gumbel_table():
    n = _ROWS * _VOCAB
    o0, o1 = _threefry2x32(
        0, 42, np.zeros(n, dtype=np.uint32), np.arange(n, dtype=np.uint32)
    )
    bits = (o0 ^ o1).astype(np.uint32)
    fb = ((bits >> np.uint32(9)) | np.uint32(0x3F800000)).astype(np.uint32)
    u = fb.view(np.float32).astype(np.float64) - 1.0
    e = (-np.log1p(-u)).astype(np.float32)
    e = np.maximum(e, np.float32(1e-10))
    g = (-np.log(e.astype(np.float64))).astype(np.float32)
    return g.reshape(_ROWS, _VOCAB)


_GUMBEL = _gumbel_table()


_RB = 8
_LANES = 128
_K = 4
_CHUNK = _K * _LANES
_NITER = 195
_TAIL0 = _NITER * _CHUNK
_BIG = 2**30


def _sample_body(t_ref, x_ref, g_ref, o_ref):
    l = x_ref[...] / t_ref[...]
    s = l + g_ref[...]
    col = jax.lax.broadcasted_iota(jnp.int32, s.shape, 1)
    s = jnp.where(col < _VOCAB, s, -jnp.inf)
    idx = jnp.argmax(s, axis=1)
    o_ref[...] = idx[:, None].astype(jnp.int32)


def kernel(logits, temperatures):
    rb = 16
    grid = (_ROWS // rb,)
    out = pl.pallas_call(
        _sample_body,
        grid=grid,
        in_specs=[
            pl.BlockSpec((rb, 1), lambda i: (i, 0)),
            pl.BlockSpec((rb, _VOCAB), lambda i: (i, 0)),
            pl.BlockSpec((rb, _VOCAB), lambda i: (i, 0)),
        ],
        out_specs=pl.BlockSpec((rb, 1), lambda i: (i, 0)),
        out_shape=jax.ShapeDtypeStruct((_ROWS, 1), jnp.int32),
    )(temperatures[:, None], logits, jnp.asarray(_GUMBEL))
    return out.reshape(_ROWS)
